# Initial kernel scaffold; baseline (speedup 1.0000x reference)
import jax
import jax.numpy as jnp
from jax.experimental import pallas as pl


def kernel(feature, boxes, w1a, b1a, w1b, b1b, w2a, b2a, w2b, b2b, w3a, b3a, w3b, b3b, wDa, bDa, wDb, bDb, wF, bF):
    raise NotImplementedError("write your pallas kernel here")


# speedup vs baseline: 1.1688x; 1.1688x over previous
import jax
import jax.numpy as jnp
from jax.experimental import pallas as pl
from jax.experimental.pallas import tpu as pltpu

A, K, CIN, H, W = 8, 50, 64, 100, 352
STRIP = 88
NS = W // STRIP
BW = 104
BR = 104
FLAT = BR * BW
OUT0 = 2 * BW
NOUT = 100 * BW


def _kernel(feat_hbm, u_ref, d_ref, l_ref, r_ref, inva_ref, hh_ref, ww_ref,
            w1a, b1a, w1b, b1b, w2a, b2a, w2b, b2b,
            w3a, b3a, w3b, b3b, wDa, bDa, wDb, bDb, wF, bF,
            out_ref, slab, bufA, bufB, sem):
    a = pl.program_id(0)
    s = pl.program_id(1)

    cp = pltpu.make_async_copy(
        feat_hbm.at[a, :, pl.ds(pl.multiple_of(s * STRIP, 8), BW), :],
        slab.at[1:103, :, :], sem)
    cp.start()

    slab[0:1] = jnp.zeros((1, BW, CIN), jnp.bfloat16)
    slab[103:104] = jnp.zeros((1, BW, CIN), jnp.bfloat16)
    zpad = jnp.zeros((OUT0, 256), jnp.bfloat16)
    bufA[0:OUT0, :] = zpad
    bufA[OUT0 + NOUT:FLAT, :] = zpad
    bufB[0:OUT0, :] = zpad
    bufB[OUT0 + NOUT:FLAT, :] = zpad

    cp.wait()
    x0 = slab[...].reshape(FLAT, CIN)

    def conv3(x_get, cin, wt, bias, out_buf, cout):
        acc = None
        for t in range(9):
            dy, dx = divmod(t, 3)
            start = 103 + dy * BW + dx
            lhs = x_get(start, cin)
            p = jax.lax.dot_general(lhs, wt[t], (((1,), (0,)), ((), ())),
                                    preferred_element_type=jnp.float32)
            acc = p if acc is None else acc + p
        y = jnp.maximum(acc + bias[...], 0.0)
        out_buf[OUT0:OUT0 + NOUT, :cout] = y.astype(jnp.bfloat16)

    def from_val(v):
        return lambda start, cin: jax.lax.slice(v, (start, 0),
                                                (start + NOUT, cin))

    def from_ref(ref):
        return lambda start, cin: ref[start:start + NOUT, :cin]

    conv3(from_val(x0), CIN, w1a, b1a, bufA, 128)
    conv3(from_ref(bufA), 128, w1b, b1b, bufB, 128)
    conv3(from_ref(bufB), 128, w2a, b2a, bufA, 128)
    conv3(from_ref(bufA), 128, w2b, b2b, bufB, 128)
    conv3(from_ref(bufB), 128, w3a, b3a, bufA, 256)
    conv3(from_ref(bufA), 256, w3b, b3b, bufB, 256)
    conv3(from_ref(bufB), 256, wDa, bDa, bufA, 256)

    x7 = bufA[OUT0:OUT0 + NOUT, :].reshape(100, BW, 256)
    x7 = jax.lax.slice(x7, (0, 8, 0), (100, 96, 256)).reshape(8800, 256)
    z2 = jnp.maximum(jax.lax.dot_general(
        x7, wDb[...], (((1,), (0,)), ((), ())),
        preferred_element_type=jnp.float32) + bDb[...], 0.0).astype(jnp.bfloat16)
    z3 = jnp.maximum(jax.lax.dot_general(
        z2, wF[...], (((1,), (0,)), ((), ())),
        preferred_element_type=jnp.float32) + bF[...], 0.0).astype(jnp.bfloat16)

    hh = hh_ref[...]
    wg = ww_ref[...] + s * STRIP
    u = u_ref[0]
    d = d_ref[0]
    lo = l_ref[0]
    ro = r_ref[0]
    cond = (hh >= u) & (hh < d) & (wg >= lo) & (wg < ro)
    m = jnp.where(cond, 1.0, 0.0).astype(jnp.bfloat16)
    pooled = jax.lax.dot_general(m, z3, (((1,), (0,)), ((), ())),
                                 preferred_element_type=jnp.float32)
    pooled = pooled * inva_ref[0]

    @pl.when(s == 0)
    def _():
        out_ref[...] = pooled[None]

    @pl.when(s > 0)
    def _():
        out_ref[...] += pooled[None]


def kernel(feature, boxes, w1a, b1a, w1b, b1b, w2a, b2a, w2b, b2b,
           w3a, b3a, w3b, b3b, wDa, bDa, wDb, bDb, wF, bF):
    feat = jnp.pad(jnp.transpose(feature, (0, 2, 3, 1)),
                   ((0, 0), (1, 1), (8, 8), (0, 0))).astype(jnp.bfloat16)

    def tw(w):
        return jnp.transpose(w, (2, 3, 1, 0)).reshape(
            9, w.shape[1], w.shape[0]).astype(jnp.bfloat16)

    def tb(b):
        return b.reshape(1, -1)

    wDb2 = wDb[:, :, 0, 0].T.astype(jnp.bfloat16)
    wF2 = wF[:, :, 0, 0].T.astype(jnp.bfloat16)

    l = boxes[..., 0].reshape(A, K, 1)
    r = boxes[..., 1].reshape(A, K, 1)
    u = boxes[..., 2].reshape(A, K, 1)
    d = boxes[..., 3].reshape(A, K, 1)
    inva = 1.0 / ((d - u) * (r - l)).astype(jnp.float32)

    ji = jnp.arange(8800, dtype=jnp.int32)
    hh = (ji // STRIP).reshape(1, 8800)
    ww = (ji % STRIP).reshape(1, 8800)

    vspec = pl.BlockSpec(memory_space=pltpu.VMEM)
    desc = pl.pallas_call(
        _kernel,
        out_shape=jax.ShapeDtypeStruct((A, K, 256), jnp.float32),
        grid=(A, NS),
        in_specs=[
            pl.BlockSpec(memory_space=pl.ANY),
            pl.BlockSpec((1, K, 1), lambda a, s: (a, 0, 0)),
            pl.BlockSpec((1, K, 1), lambda a, s: (a, 0, 0)),
            pl.BlockSpec((1, K, 1), lambda a, s: (a, 0, 0)),
            pl.BlockSpec((1, K, 1), lambda a, s: (a, 0, 0)),
            pl.BlockSpec((1, K, 1), lambda a, s: (a, 0, 0)),
            pl.BlockSpec((1, 8800), lambda a, s: (0, 0)),
            pl.BlockSpec((1, 8800), lambda a, s: (0, 0)),
        ] + [pl.BlockSpec(memory_space=pltpu.VMEM)] * 18,
        out_specs=pl.BlockSpec((1, K, 256), lambda a, s: (a, 0, 0)),
        scratch_shapes=[
            pltpu.VMEM((BR, BW, CIN), jnp.bfloat16),
            pltpu.VMEM((FLAT, 256), jnp.bfloat16),
            pltpu.VMEM((FLAT, 256), jnp.bfloat16),
            pltpu.SemaphoreType.DMA,
        ],
        compiler_params=pltpu.CompilerParams(
            dimension_semantics=("parallel", "arbitrary"),
            vmem_limit_bytes=56 * 1024 * 1024,
        ),
        name="superbox_fused",
    )(feat, u, d, l, r, inva, hh, ww,
      tw(w1a), tb(b1a), tw(w1b), tb(b1b), tw(w2a), tb(b2a), tw(w2b), tb(b2b),
      tw(w3a), tb(b3a), tw(w3b), tb(b3b), tw(wDa), tb(bDa),
      wDb2, tb(bDb), wF2, tb(bF))
    return desc


# speedup vs baseline: 1.2581x; 1.0764x over previous
import jax
import jax.numpy as jnp
from jax.experimental import pallas as pl
from jax.experimental.pallas import tpu as pltpu

A, K, CIN, H, W = 8, 50, 64, 100, 352
STRIP = 88
NS = W // STRIP
BW = 104
BR = 104
FLAT = BR * BW
OUT0 = 2 * BW
NOUT = 100 * BW


def _kernel(feat_hbm, u_ref, d_ref, l_ref, r_ref, inva_ref, hh_ref, ww_ref,
            w1a, b1a, w1b, b1b, w2a, b2a, w2b, b2b,
            w3a, b3a, w3b, b3b, wDa, bDa, wDb, bDb, wF, bF,
            out_ref, slab, bufA, bufB, bufC, sem):
    a = pl.program_id(0)
    s = pl.program_id(1)

    cp = pltpu.make_async_copy(
        feat_hbm.at[a, :, pl.ds(pl.multiple_of(s * STRIP, 8), BW), :],
        slab.at[1:103, :, :], sem)
    cp.start()

    slab[0:1] = jnp.zeros((1, BW, CIN), jnp.bfloat16)
    slab[103:104] = jnp.zeros((1, BW, CIN), jnp.bfloat16)
    zpad = jnp.zeros((OUT0, 256), jnp.bfloat16)
    bufA[0:OUT0, :] = zpad
    bufA[OUT0 + NOUT:FLAT, :] = zpad
    bufB[0:OUT0, :] = zpad
    bufB[OUT0 + NOUT:FLAT, :] = zpad

    cp.wait()
    x0 = slab[...].reshape(FLAT, CIN)

    def conv3(x_get, cin, wt, bias, out_buf, cout):
        acc = None
        for t in range(9):
            dy, dx = divmod(t, 3)
            start = 103 + dy * BW + dx
            lhs = x_get(start, cin)
            p = jax.lax.dot_general(lhs, wt[t], (((1,), (0,)), ((), ())),
                                    preferred_element_type=jnp.float32)
            acc = p if acc is None else acc + p
        y = jnp.maximum(acc + bias[...], 0.0)
        out_buf[OUT0:OUT0 + NOUT, :cout] = y.astype(jnp.bfloat16)

    def from_val(v):
        return lambda start, cin: jax.lax.slice(v, (start, 0),
                                                (start + NOUT, cin))

    def from_ref(ref):
        return lambda start, cin: ref[start:start + NOUT, :cin]

    def convcat(x_get, cin, wt, bias, out_buf, cout):
        bufC[1:FLAT, 0:cin] = x_get(0, FLAT - 1, cin)
        bufC[0:FLAT, cin:2 * cin] = x_get(0, FLAT, cin)
        bufC[0:FLAT - 1, 2 * cin:3 * cin] = x_get(1, FLAT, cin)
        acc = None
        for dy in range(3):
            start = 104 + dy * BW
            lhs = bufC[start:start + NOUT, :3 * cin]
            p = jax.lax.dot_general(lhs, wt[dy], (((1,), (0,)), ((), ())),
                                    preferred_element_type=jnp.float32)
            acc = p if acc is None else acc + p
        y = jnp.maximum(acc + bias[...], 0.0)
        out_buf[OUT0:OUT0 + NOUT, :cout] = y.astype(jnp.bfloat16)

    def rng_val(v):
        return lambda lo, hi, cin: jax.lax.slice(v, (lo, 0), (hi, cin))

    def rng_ref(ref):
        return lambda lo, hi, cin: ref[lo:hi, :cin]

    convcat(rng_val(x0), CIN, w1a, b1a, bufA, 128)
    convcat(rng_ref(bufA), 128, w1b, b1b, bufB, 128)
    convcat(rng_ref(bufB), 128, w2a, b2a, bufA, 128)
    convcat(rng_ref(bufA), 128, w2b, b2b, bufB, 128)
    convcat(rng_ref(bufB), 128, w3a, b3a, bufA, 256)
    conv3(from_ref(bufA), 256, w3b, b3b, bufB, 256)
    conv3(from_ref(bufB), 256, wDa, bDa, bufA, 256)

    x7 = bufA[OUT0:OUT0 + NOUT, :].reshape(100, BW, 256)
    x7 = jax.lax.slice(x7, (0, 8, 0), (100, 96, 256)).reshape(8800, 256)
    z2 = jnp.maximum(jax.lax.dot_general(
        x7, wDb[...], (((1,), (0,)), ((), ())),
        preferred_element_type=jnp.float32) + bDb[...], 0.0).astype(jnp.bfloat16)
    z3 = jnp.maximum(jax.lax.dot_general(
        z2, wF[...], (((1,), (0,)), ((), ())),
        preferred_element_type=jnp.float32) + bF[...], 0.0).astype(jnp.bfloat16)

    hh = hh_ref[...]
    wg = ww_ref[...] + s * STRIP
    u = u_ref[0]
    d = d_ref[0]
    lo = l_ref[0]
    ro = r_ref[0]
    cond = (hh >= u) & (hh < d) & (wg >= lo) & (wg < ro)
    m = jnp.where(cond, 1.0, 0.0).astype(jnp.bfloat16)
    pooled = jax.lax.dot_general(m, z3, (((1,), (0,)), ((), ())),
                                 preferred_element_type=jnp.float32)
    pooled = pooled * inva_ref[0]

    @pl.when(s == 0)
    def _():
        out_ref[...] = pooled[None]

    @pl.when(s > 0)
    def _():
        out_ref[...] += pooled[None]


def kernel(feature, boxes, w1a, b1a, w1b, b1b, w2a, b2a, w2b, b2b,
           w3a, b3a, w3b, b3b, wDa, bDa, wDb, bDb, wF, bF):
    feat = jnp.pad(jnp.transpose(feature, (0, 2, 3, 1)),
                   ((0, 0), (1, 1), (8, 8), (0, 0))).astype(jnp.bfloat16)

    def tw(w):
        return jnp.transpose(w, (2, 3, 1, 0)).reshape(
            9, w.shape[1], w.shape[0]).astype(jnp.bfloat16)

    def tw3(w):
        return jnp.transpose(w, (2, 3, 1, 0)).reshape(
            3, 3 * w.shape[1], w.shape[0]).astype(jnp.bfloat16)

    def tb(b):
        return b.reshape(1, -1)

    wDb2 = wDb[:, :, 0, 0].T.astype(jnp.bfloat16)
    wF2 = wF[:, :, 0, 0].T.astype(jnp.bfloat16)

    l = boxes[..., 0].reshape(A, K, 1)
    r = boxes[..., 1].reshape(A, K, 1)
    u = boxes[..., 2].reshape(A, K, 1)
    d = boxes[..., 3].reshape(A, K, 1)
    inva = 1.0 / ((d - u) * (r - l)).astype(jnp.float32)

    ji = jnp.arange(8800, dtype=jnp.int32)
    hh = (ji // STRIP).reshape(1, 8800)
    ww = (ji % STRIP).reshape(1, 8800)

    vspec = pl.BlockSpec(memory_space=pltpu.VMEM)
    desc = pl.pallas_call(
        _kernel,
        out_shape=jax.ShapeDtypeStruct((A, K, 256), jnp.float32),
        grid=(A, NS),
        in_specs=[
            pl.BlockSpec(memory_space=pl.ANY),
            pl.BlockSpec((1, K, 1), lambda a, s: (a, 0, 0)),
            pl.BlockSpec((1, K, 1), lambda a, s: (a, 0, 0)),
            pl.BlockSpec((1, K, 1), lambda a, s: (a, 0, 0)),
            pl.BlockSpec((1, K, 1), lambda a, s: (a, 0, 0)),
            pl.BlockSpec((1, K, 1), lambda a, s: (a, 0, 0)),
            pl.BlockSpec((1, 8800), lambda a, s: (0, 0)),
            pl.BlockSpec((1, 8800), lambda a, s: (0, 0)),
        ] + [pl.BlockSpec(memory_space=pltpu.VMEM)] * 18,
        out_specs=pl.BlockSpec((1, K, 256), lambda a, s: (a, 0, 0)),
        scratch_shapes=[
            pltpu.VMEM((BR, BW, CIN), jnp.bfloat16),
            pltpu.VMEM((FLAT, 256), jnp.bfloat16),
            pltpu.VMEM((FLAT, 256), jnp.bfloat16),
            pltpu.VMEM((FLAT, 384), jnp.bfloat16),
            pltpu.SemaphoreType.DMA,
        ],
        compiler_params=pltpu.CompilerParams(
            dimension_semantics=("parallel", "arbitrary"),
            vmem_limit_bytes=56 * 1024 * 1024,
        ),
        name="superbox_fused",
    )(feat, u, d, l, r, inva, hh, ww,
      tw3(w1a), tb(b1a), tw3(w1b), tb(b1b), tw3(w2a), tb(b2a), tw3(w2b), tb(b2b),
      tw3(w3a), tb(b3a), tw(w3b), tb(b3b), tw(wDa), tb(bDa),
      wDb2, tb(bDb), wF2, tb(bF))
    return desc


# speedup vs baseline: 1.3046x; 1.0370x over previous
import jax
import jax.numpy as jnp
from jax.experimental import pallas as pl
from jax.experimental.pallas import tpu as pltpu

A, K, CIN, H, W = 8, 50, 64, 100, 352
STRIP = 88
NS = W // STRIP
BW = 104
BR = 104
FLAT = BR * BW
OUT0 = 2 * BW
NOUT = 100 * BW


def _kernel(feat_hbm, u_ref, d_ref, l_ref, r_ref, inva_ref, hh_ref, ww_ref,
            w1a, b1a, w1b, b1b, w2a, b2a, w2b, b2b,
            w3a, b3a, w3b, b3b, wDa, bDa, wDb, bDb, wF, bF,
            out_ref, slab, bufA, bufB, bufC, sem):
    a = pl.program_id(0)
    s = pl.program_id(1)

    cp = pltpu.make_async_copy(
        feat_hbm.at[a, :, pl.ds(pl.multiple_of(s * STRIP, 8), BW), :],
        slab.at[1:103, :, :], sem)
    cp.start()

    slab[0:1] = jnp.zeros((1, BW, CIN), jnp.bfloat16)
    slab[103:104] = jnp.zeros((1, BW, CIN), jnp.bfloat16)
    zpad = jnp.zeros((OUT0, 256), jnp.bfloat16)
    bufA[0:OUT0, :] = zpad
    bufA[OUT0 + NOUT:FLAT, :] = zpad
    bufB[0:OUT0, :] = zpad
    bufB[OUT0 + NOUT:FLAT, :] = zpad

    cp.wait()
    x0 = slab[...].reshape(FLAT, CIN)

    def conv3(x_get, cin, wt, bias, out_buf, cout):
        acc = None
        for t in range(9):
            dy, dx = divmod(t, 3)
            start = 103 + dy * BW + dx
            lhs = x_get(start, cin)
            p = jax.lax.dot_general(lhs, wt[t], (((1,), (0,)), ((), ())),
                                    preferred_element_type=jnp.float32)
            acc = p if acc is None else acc + p
        y = jnp.maximum(acc + bias[...], 0.0)
        out_buf[OUT0:OUT0 + NOUT, :cout] = y.astype(jnp.bfloat16)

    def from_val(v):
        return lambda start, cin: jax.lax.slice(v, (start, 0),
                                                (start + NOUT, cin))

    def from_ref(ref):
        return lambda start, cin: ref[start:start + NOUT, :cin]

    def convcat(x_get, cin, wt, bias, out_buf, cout):
        bufC[1:FLAT, 0:cin] = x_get(0, FLAT - 1, cin)
        bufC[0:FLAT, cin:2 * cin] = x_get(0, FLAT, cin)
        bufC[0:FLAT - 1, 2 * cin:3 * cin] = x_get(1, FLAT, cin)
        acc = None
        for dy in range(3):
            start = 104 + dy * BW
            lhs = bufC[start:start + NOUT, :3 * cin]
            p = jax.lax.dot_general(lhs, wt[dy], (((1,), (0,)), ((), ())),
                                    preferred_element_type=jnp.float32)
            acc = p if acc is None else acc + p
        y = jnp.maximum(acc + bias[...], 0.0)
        out_buf[OUT0:OUT0 + NOUT, :cout] = y.astype(jnp.bfloat16)

    def convstk(x_get, cin, wt, bias, out_buf, cout):
        bufC[1:FLAT, 0:cin] = x_get(0, FLAT - 1, cin)
        bufC[0:FLAT, cin:2 * cin] = x_get(0, FLAT, cin)
        bufC[0:FLAT - 1, 2 * cin:3 * cin] = x_get(1, FLAT, cin)
        mext = NOUT + 2 * BW
        lhs = bufC[BW:BW + mext, :3 * cin]
        p = jax.lax.dot_general(lhs, wt[...], (((1,), (0,)), ((), ())),
                                preferred_element_type=jnp.float32)
        acc = (jax.lax.slice(p, (0, 0), (NOUT, cout))
               + jax.lax.slice(p, (BW, cout), (BW + NOUT, 2 * cout))
               + jax.lax.slice(p, (2 * BW, 2 * cout), (2 * BW + NOUT, 3 * cout)))
        y = jnp.maximum(acc + bias[...], 0.0)
        out_buf[OUT0:OUT0 + NOUT, :cout] = y.astype(jnp.bfloat16)

    def rng_val(v):
        return lambda lo, hi, cin: jax.lax.slice(v, (lo, 0), (hi, cin))

    def rng_ref(ref):
        return lambda lo, hi, cin: ref[lo:hi, :cin]

    convstk(rng_val(x0), CIN, w1a, b1a, bufA, 128)
    convstk(rng_ref(bufA), 128, w1b, b1b, bufB, 128)
    convstk(rng_ref(bufB), 128, w2a, b2a, bufA, 128)
    convstk(rng_ref(bufA), 128, w2b, b2b, bufB, 128)
    convcat(rng_ref(bufB), 128, w3a, b3a, bufA, 256)
    conv3(from_ref(bufA), 256, w3b, b3b, bufB, 256)
    conv3(from_ref(bufB), 256, wDa, bDa, bufA, 256)

    x7 = bufA[OUT0:OUT0 + NOUT, :].reshape(100, BW, 256)
    x7 = jax.lax.slice(x7, (0, 8, 0), (100, 96, 256)).reshape(8800, 256)
    z2 = jnp.maximum(jax.lax.dot_general(
        x7, wDb[...], (((1,), (0,)), ((), ())),
        preferred_element_type=jnp.float32) + bDb[...], 0.0).astype(jnp.bfloat16)
    z3 = jnp.maximum(jax.lax.dot_general(
        z2, wF[...], (((1,), (0,)), ((), ())),
        preferred_element_type=jnp.float32) + bF[...], 0.0).astype(jnp.bfloat16)

    hh = hh_ref[...]
    wg = ww_ref[...] + s * STRIP
    u = u_ref[0]
    d = d_ref[0]
    lo = l_ref[0]
    ro = r_ref[0]
    cond = (hh >= u) & (hh < d) & (wg >= lo) & (wg < ro)
    m = jnp.where(cond, 1.0, 0.0).astype(jnp.bfloat16)
    pooled = jax.lax.dot_general(m, z3, (((1,), (0,)), ((), ())),
                                 preferred_element_type=jnp.float32)
    pooled = pooled * inva_ref[0]

    @pl.when(s == 0)
    def _():
        out_ref[...] = pooled[None]

    @pl.when(s > 0)
    def _():
        out_ref[...] += pooled[None]


def kernel(feature, boxes, w1a, b1a, w1b, b1b, w2a, b2a, w2b, b2b,
           w3a, b3a, w3b, b3b, wDa, bDa, wDb, bDb, wF, bF):
    feat = jnp.pad(jnp.transpose(feature, (0, 2, 3, 1)),
                   ((0, 0), (1, 1), (8, 8), (0, 0))).astype(jnp.bfloat16)

    def tw(w):
        return jnp.transpose(w, (2, 3, 1, 0)).reshape(
            9, w.shape[1], w.shape[0]).astype(jnp.bfloat16)

    def tw3(w):
        return jnp.transpose(w, (2, 3, 1, 0)).reshape(
            3, 3 * w.shape[1], w.shape[0]).astype(jnp.bfloat16)

    def tws(w):
        t = jnp.transpose(w, (2, 3, 1, 0)).reshape(3, 3 * w.shape[1], w.shape[0])
        return jnp.transpose(t, (1, 0, 2)).reshape(
            3 * w.shape[1], 3 * w.shape[0]).astype(jnp.bfloat16)

    def tb(b):
        return b.reshape(1, -1)

    wDb2 = wDb[:, :, 0, 0].T.astype(jnp.bfloat16)
    wF2 = wF[:, :, 0, 0].T.astype(jnp.bfloat16)

    l = boxes[..., 0].reshape(A, K, 1)
    r = boxes[..., 1].reshape(A, K, 1)
    u = boxes[..., 2].reshape(A, K, 1)
    d = boxes[..., 3].reshape(A, K, 1)
    inva = 1.0 / ((d - u) * (r - l)).astype(jnp.float32)

    ji = jnp.arange(8800, dtype=jnp.int32)
    hh = (ji // STRIP).reshape(1, 8800)
    ww = (ji % STRIP).reshape(1, 8800)

    vspec = pl.BlockSpec(memory_space=pltpu.VMEM)
    desc = pl.pallas_call(
        _kernel,
        out_shape=jax.ShapeDtypeStruct((A, K, 256), jnp.float32),
        grid=(A, NS),
        in_specs=[
            pl.BlockSpec(memory_space=pl.ANY),
            pl.BlockSpec((1, K, 1), lambda a, s: (a, 0, 0)),
            pl.BlockSpec((1, K, 1), lambda a, s: (a, 0, 0)),
            pl.BlockSpec((1, K, 1), lambda a, s: (a, 0, 0)),
            pl.BlockSpec((1, K, 1), lambda a, s: (a, 0, 0)),
            pl.BlockSpec((1, K, 1), lambda a, s: (a, 0, 0)),
            pl.BlockSpec((1, 8800), lambda a, s: (0, 0)),
            pl.BlockSpec((1, 8800), lambda a, s: (0, 0)),
        ] + [pl.BlockSpec(memory_space=pltpu.VMEM)] * 18,
        out_specs=pl.BlockSpec((1, K, 256), lambda a, s: (a, 0, 0)),
        scratch_shapes=[
            pltpu.VMEM((BR, BW, CIN), jnp.bfloat16),
            pltpu.VMEM((FLAT, 256), jnp.bfloat16),
            pltpu.VMEM((FLAT, 256), jnp.bfloat16),
            pltpu.VMEM((FLAT, 384), jnp.bfloat16),
            pltpu.SemaphoreType.DMA,
        ],
        compiler_params=pltpu.CompilerParams(
            dimension_semantics=("parallel", "arbitrary"),
            vmem_limit_bytes=56 * 1024 * 1024,
        ),
        name="superbox_fused",
    )(feat, u, d, l, r, inva, hh, ww,
      tws(w1a), tb(b1a), tws(w1b), tb(b1b), tws(w2a), tb(b2a), tws(w2b), tb(b2b),
      tw3(w3a), tb(b3a), tw(w3b), tb(b3b), tw(wDa), tb(bDa),
      wDb2, tb(bDb), wF2, tb(bF))
    return desc


# speedup vs baseline: 1.3907x; 1.0660x over previous
import jax
import jax.numpy as jnp
from jax.experimental import pallas as pl
from jax.experimental.pallas import tpu as pltpu

A, K, CIN, H, W = 8, 50, 64, 100, 352
STRIP = 88
NS = W // STRIP
BW = 104
BR = 104
FLAT = BR * BW
OUT0 = 2 * BW
NOUT = 100 * BW


def _kernel(feat_hbm, u_ref, d_ref, l_ref, r_ref, inva_ref, hh_ref, ww_ref,
            w1a, b1a, w1b, b1b, w2a, b2a, w2b, b2b,
            w3a, b3a, w3b, b3b, wDa, bDa, wDb, bDb, wF, bF,
            out_ref, slab, bufA, bufB, bufC, sem):
    a = pl.program_id(0)
    s = pl.program_id(1)

    cp = pltpu.make_async_copy(
        feat_hbm.at[a, :, pl.ds(pl.multiple_of(s * STRIP, 8), BW), :],
        slab.at[1:103, :, :], sem)
    cp.start()

    slab[0:1] = jnp.zeros((1, BW, CIN), jnp.bfloat16)
    slab[103:104] = jnp.zeros((1, BW, CIN), jnp.bfloat16)
    zpad = jnp.zeros((OUT0, 256), jnp.bfloat16)
    bufA[0:OUT0, :] = zpad
    bufA[OUT0 + NOUT:FLAT, :] = zpad
    bufB[0:OUT0, :] = zpad
    bufB[OUT0 + NOUT:FLAT, :] = zpad

    cp.wait()
    x0 = slab[...].reshape(FLAT, CIN)

    def conv3(x_get, cin, wt, bias, out_buf, cout):
        acc = None
        for t in range(9):
            dy, dx = divmod(t, 3)
            start = 103 + dy * BW + dx
            lhs = x_get(start, cin)
            p = jax.lax.dot_general(lhs, wt[t], (((1,), (0,)), ((), ())),
                                    preferred_element_type=jnp.float32)
            acc = p if acc is None else acc + p
        y = jnp.maximum(acc + bias[...], 0.0)
        out_buf[OUT0:OUT0 + NOUT, :cout] = y.astype(jnp.bfloat16)

    def from_val(v):
        return lambda start, cin: jax.lax.slice(v, (start, 0),
                                                (start + NOUT, cin))

    def from_ref(ref):
        return lambda start, cin: ref[start:start + NOUT, :cin]

    def convcat(x_get, cin, wt, bias, out_buf, cout):
        bufC[1:FLAT, 0:cin] = x_get(0, FLAT - 1, cin)
        bufC[0:FLAT, cin:2 * cin] = x_get(0, FLAT, cin)
        bufC[0:FLAT - 1, 2 * cin:3 * cin] = x_get(1, FLAT, cin)
        acc = None
        for dy in range(3):
            start = 104 + dy * BW
            lhs = bufC[start:start + NOUT, :3 * cin]
            p = jax.lax.dot_general(lhs, wt[dy], (((1,), (0,)), ((), ())),
                                    preferred_element_type=jnp.float32)
            acc = p if acc is None else acc + p
        y = jnp.maximum(acc + bias[...], 0.0)
        out_buf[OUT0:OUT0 + NOUT, :cout] = y.astype(jnp.bfloat16)

    def convstk(x_get, cin, wt, bias, out_buf, cout):
        bufC[1:FLAT, 0:cin] = x_get(0, FLAT - 1, cin)
        bufC[0:FLAT, cin:2 * cin] = x_get(0, FLAT, cin)
        bufC[0:FLAT - 1, 2 * cin:3 * cin] = x_get(1, FLAT, cin)
        half = NOUT // 2
        for c0 in (0, half):
            lhs = bufC[BW + c0:BW + c0 + half + 2 * BW, :3 * cin]
            p = jax.lax.dot_general(lhs, wt[...], (((1,), (0,)), ((), ())),
                                    preferred_element_type=jnp.float32)
            acc = (jax.lax.slice(p, (0, 0), (half, cout))
                   + jax.lax.slice(p, (BW, cout), (BW + half, 2 * cout))
                   + jax.lax.slice(p, (2 * BW, 2 * cout), (2 * BW + half, 3 * cout)))
            y = jnp.maximum(acc + bias[...], 0.0)
            out_buf[OUT0 + c0:OUT0 + c0 + half, :cout] = y.astype(jnp.bfloat16)

    def rng_val(v):
        return lambda lo, hi, cin: jax.lax.slice(v, (lo, 0), (hi, cin))

    def rng_ref(ref):
        return lambda lo, hi, cin: ref[lo:hi, :cin]

    convstk(rng_val(x0), CIN, w1a, b1a, bufA, 128)
    convstk(rng_ref(bufA), 128, w1b, b1b, bufB, 128)
    convstk(rng_ref(bufB), 128, w2a, b2a, bufA, 128)
    convstk(rng_ref(bufA), 128, w2b, b2b, bufB, 128)
    convcat(rng_ref(bufB), 128, w3a, b3a, bufA, 256)
    convcat(rng_ref(bufA), 256, w3b, b3b, bufB, 256)
    convcat(rng_ref(bufB), 256, wDa, bDa, bufA, 256)

    x7 = bufA[OUT0:OUT0 + NOUT, :]
    z2 = jnp.maximum(jax.lax.dot_general(
        x7, wDb[...], (((1,), (0,)), ((), ())),
        preferred_element_type=jnp.float32) + bDb[...], 0.0).astype(jnp.bfloat16)
    z3 = jnp.maximum(jax.lax.dot_general(
        z2, wF[...], (((1,), (0,)), ((), ())),
        preferred_element_type=jnp.float32) + bF[...], 0.0).astype(jnp.bfloat16)

    hh = hh_ref[...]
    wg = ww_ref[...] + s * STRIP
    u = u_ref[0]
    d = d_ref[0]
    lo = l_ref[0]
    ro = r_ref[0]
    cond = (hh >= u) & (hh < d) & (wg >= lo) & (wg < ro)
    m = jnp.where(cond, 1.0, 0.0).astype(jnp.bfloat16)
    pooled = jax.lax.dot_general(m, z3, (((1,), (0,)), ((), ())),
                                 preferred_element_type=jnp.float32)
    pooled = pooled * inva_ref[0]

    @pl.when(s == 0)
    def _():
        out_ref[...] = pooled[None]

    @pl.when(s > 0)
    def _():
        out_ref[...] += pooled[None]


def kernel(feature, boxes, w1a, b1a, w1b, b1b, w2a, b2a, w2b, b2b,
           w3a, b3a, w3b, b3b, wDa, bDa, wDb, bDb, wF, bF):
    feat = jnp.pad(jnp.transpose(feature, (0, 2, 3, 1)),
                   ((0, 0), (1, 1), (8, 8), (0, 0))).astype(jnp.bfloat16)

    def tw(w):
        return jnp.transpose(w, (2, 3, 1, 0)).reshape(
            9, w.shape[1], w.shape[0]).astype(jnp.bfloat16)

    def tw3(w):
        return jnp.transpose(w, (2, 3, 1, 0)).reshape(
            3, 3 * w.shape[1], w.shape[0]).astype(jnp.bfloat16)

    def tws(w):
        t = jnp.transpose(w, (2, 3, 1, 0)).reshape(3, 3 * w.shape[1], w.shape[0])
        return jnp.transpose(t, (1, 0, 2)).reshape(
            3 * w.shape[1], 3 * w.shape[0]).astype(jnp.bfloat16)

    def tb(b):
        return b.reshape(1, -1)

    wDb2 = wDb[:, :, 0, 0].T.astype(jnp.bfloat16)
    wF2 = wF[:, :, 0, 0].T.astype(jnp.bfloat16)

    l = boxes[..., 0].reshape(A, K, 1)
    r = boxes[..., 1].reshape(A, K, 1)
    u = boxes[..., 2].reshape(A, K, 1)
    d = boxes[..., 3].reshape(A, K, 1)
    inva = 1.0 / ((d - u) * (r - l)).astype(jnp.float32)

    ji = jnp.arange(NOUT, dtype=jnp.int32)
    col = ji % BW
    central = (col >= 8) & (col < 96)
    hh = jnp.where(central, ji // BW, -1).reshape(1, NOUT).astype(jnp.int32)
    ww = (col - 8).reshape(1, NOUT).astype(jnp.int32)

    vspec = pl.BlockSpec(memory_space=pltpu.VMEM)
    desc = pl.pallas_call(
        _kernel,
        out_shape=jax.ShapeDtypeStruct((A, K, 256), jnp.float32),
        grid=(A, NS),
        in_specs=[
            pl.BlockSpec(memory_space=pl.ANY),
            pl.BlockSpec((1, K, 1), lambda a, s: (a, 0, 0)),
            pl.BlockSpec((1, K, 1), lambda a, s: (a, 0, 0)),
            pl.BlockSpec((1, K, 1), lambda a, s: (a, 0, 0)),
            pl.BlockSpec((1, K, 1), lambda a, s: (a, 0, 0)),
            pl.BlockSpec((1, K, 1), lambda a, s: (a, 0, 0)),
            pl.BlockSpec((1, NOUT), lambda a, s: (0, 0)),
            pl.BlockSpec((1, NOUT), lambda a, s: (0, 0)),
        ] + [pl.BlockSpec(memory_space=pltpu.VMEM)] * 18,
        out_specs=pl.BlockSpec((1, K, 256), lambda a, s: (a, 0, 0)),
        scratch_shapes=[
            pltpu.VMEM((BR, BW, CIN), jnp.bfloat16),
            pltpu.VMEM((FLAT, 256), jnp.bfloat16),
            pltpu.VMEM((FLAT, 256), jnp.bfloat16),
            pltpu.VMEM((FLAT, 768), jnp.bfloat16),
            pltpu.SemaphoreType.DMA,
        ],
        compiler_params=pltpu.CompilerParams(
            dimension_semantics=("parallel", "arbitrary"),
            vmem_limit_bytes=56 * 1024 * 1024,
        ),
        name="superbox_fused",
    )(feat, u, d, l, r, inva, hh, ww,
      tws(w1a), tb(b1a), tws(w1b), tb(b1b), tws(w2a), tb(b2a), tws(w2b), tb(b2b),
      tw3(w3a), tb(b3a), tw3(w3b), tb(b3b), tw3(wDa), tb(bDa),
      wDb2, tb(bDb), wF2, tb(bF))
    return desc


# speedup vs baseline: 1.7706x; 1.2732x over previous
import jax
import jax.numpy as jnp
from jax.experimental import pallas as pl
from jax.experimental.pallas import tpu as pltpu

A, K, CIN, H, W = 8, 50, 64, 100, 352
STRIP = 88
NS = W // STRIP
BW = 104
BR = 104
FLAT = BR * BW
OUT0 = 2 * BW
NOUT = 100 * BW


def _kernel(feat_hbm, u_ref, d_ref, l_ref, r_ref, inva_ref, hh_ref, ww_ref,
            w1a, b1a, w1b, b1b, w2a, b2a, w2b, b2b,
            w3a, b3a, w3b, b3b, wDa, bDa, wDb, bDb, wF, bF,
            out_ref, slab, bufA, bufB, bufC, sem):
    a = pl.program_id(0)
    s = pl.program_id(1)

    cp = pltpu.make_async_copy(
        feat_hbm.at[a, :, pl.ds(pl.multiple_of(s * STRIP, 8), BW), :],
        slab.at[1:103, :, :], sem)
    cp.start()

    slab[0:1] = jnp.zeros((1, BW, CIN), jnp.bfloat16)
    slab[103:104] = jnp.zeros((1, BW, CIN), jnp.bfloat16)
    zpad = jnp.zeros((OUT0, 256), jnp.bfloat16)
    bufA[0:OUT0, :] = zpad
    bufA[OUT0 + NOUT:FLAT, :] = zpad
    bufB[0:OUT0, :] = zpad
    bufB[OUT0 + NOUT:FLAT, :] = zpad

    cp.wait()
    x0 = slab[...].reshape(FLAT, CIN)

    def conv3(x_get, cin, wt, bias, out_buf, cout):
        acc = None
        for t in range(9):
            dy, dx = divmod(t, 3)
            start = BW - 1 + dy * BW + dx
            lhs = x_get(start, cin)
            p = jax.lax.dot_general(lhs, wt[t], (((1,), (0,)), ((), ())),
                                    preferred_element_type=jnp.float32)
            acc = p if acc is None else acc + p
        y = jnp.maximum(acc + bias[...], 0.0)
        out_buf[OUT0:OUT0 + NOUT, :cout] = y.astype(jnp.bfloat16)

    def from_val(v):
        return lambda start, cin: jax.lax.slice(v, (start, 0),
                                                (start + NOUT, cin))

    def from_ref(ref):
        return lambda start, cin: ref[start:start + NOUT, :cin]

    def convcat(x_get, cin, wt, bias, out_buf, cout):
        bufC[1:FLAT, 0:cin] = x_get(0, FLAT - 1, cin)
        bufC[0:FLAT, cin:2 * cin] = x_get(0, FLAT, cin)
        bufC[0:FLAT - 1, 2 * cin:3 * cin] = x_get(1, FLAT, cin)
        half = NOUT // 2
        for c0 in (0, half):
            acc = None
            for dy in range(3):
                start = BW + dy * BW + c0
                lhs = bufC[start:start + half, :3 * cin]
                p = jax.lax.dot_general(lhs, wt[dy], (((1,), (0,)), ((), ())),
                                        preferred_element_type=jnp.float32)
                acc = p if acc is None else acc + p
            y = jnp.maximum(acc + bias[...], 0.0)
            out_buf[OUT0 + c0:OUT0 + c0 + half, :cout] = y.astype(jnp.bfloat16)

    def convstk(x_get, cin, wt, bias, out_buf, cout):
        bufC[1:FLAT, 0:cin] = x_get(0, FLAT - 1, cin)
        bufC[0:FLAT, cin:2 * cin] = x_get(0, FLAT, cin)
        bufC[0:FLAT - 1, 2 * cin:3 * cin] = x_get(1, FLAT, cin)
        half = NOUT // 2
        for c0 in (0, half):
            lhs = bufC[BW + c0:BW + c0 + half + 2 * BW, :3 * cin]
            p = jax.lax.dot_general(lhs, wt[...], (((1,), (0,)), ((), ())),
                                    preferred_element_type=jnp.float32)
            acc = (jax.lax.slice(p, (0, 0), (half, cout))
                   + jax.lax.slice(p, (BW, cout), (BW + half, 2 * cout))
                   + jax.lax.slice(p, (2 * BW, 2 * cout), (2 * BW + half, 3 * cout)))
            y = jnp.maximum(acc + bias[...], 0.0)
            out_buf[OUT0 + c0:OUT0 + c0 + half, :cout] = y.astype(jnp.bfloat16)

    def rng_val(v):
        return lambda lo, hi, cin: jax.lax.slice(v, (lo, 0), (hi, cin))

    def rng_ref(ref):
        return lambda lo, hi, cin: ref[lo:hi, :cin]

    convstk(rng_val(x0), CIN, w1a, b1a, bufA, 128)
    convstk(rng_ref(bufA), 128, w1b, b1b, bufB, 128)
    convstk(rng_ref(bufB), 128, w2a, b2a, bufA, 128)
    convstk(rng_ref(bufA), 128, w2b, b2b, bufB, 128)
    convcat(rng_ref(bufB), 128, w3a, b3a, bufA, 256)
    convcat(rng_ref(bufA), 256, w3b, b3b, bufB, 256)
    convcat(rng_ref(bufB), 256, wDa, bDa, bufA, 256)

    u = u_ref[0]
    d = d_ref[0]
    lo = l_ref[0]
    ro = r_ref[0]
    half = NOUT // 2
    pooled = None
    for c0 in (0, half):
        x7 = bufA[OUT0 + c0:OUT0 + c0 + half, :]
        z2 = jnp.maximum(jax.lax.dot_general(
            x7, wDb[...], (((1,), (0,)), ((), ())),
            preferred_element_type=jnp.float32) + bDb[...], 0.0).astype(jnp.bfloat16)
        z3 = jnp.maximum(jax.lax.dot_general(
            z2, wF[...], (((1,), (0,)), ((), ())),
            preferred_element_type=jnp.float32) + bF[...], 0.0).astype(jnp.bfloat16)
        hh = hh_ref[:, c0:c0 + half]
        wg = ww_ref[:, c0:c0 + half] + s * STRIP
        cond = (hh >= u) & (hh < d) & (wg >= lo) & (wg < ro)
        m = jnp.where(cond, 1.0, 0.0).astype(jnp.bfloat16)
        pc = jax.lax.dot_general(m, z3, (((1,), (0,)), ((), ())),
                                 preferred_element_type=jnp.float32)
        pooled = pc if pooled is None else pooled + pc
    pooled = pooled * inva_ref[0]

    @pl.when(s == 0)
    def _():
        out_ref[...] = pooled[None]

    @pl.when(s > 0)
    def _():
        out_ref[...] += pooled[None]


def kernel(feature, boxes, w1a, b1a, w1b, b1b, w2a, b2a, w2b, b2b,
           w3a, b3a, w3b, b3b, wDa, bDa, wDb, bDb, wF, bF):
    feat = jnp.pad(jnp.transpose(feature, (0, 2, 3, 1)),
                   ((0, 0), (1, 1), (8, BW - 88 - 8), (0, 0))).astype(jnp.bfloat16)

    def tw(w):
        return jnp.transpose(w, (2, 3, 1, 0)).reshape(
            9, w.shape[1], w.shape[0]).astype(jnp.bfloat16)

    def tw3(w):
        return jnp.transpose(w, (2, 3, 1, 0)).reshape(
            3, 3 * w.shape[1], w.shape[0]).astype(jnp.bfloat16)

    def tws(w):
        t = jnp.transpose(w, (2, 3, 1, 0)).reshape(3, 3 * w.shape[1], w.shape[0])
        return jnp.transpose(t, (1, 0, 2)).reshape(
            3 * w.shape[1], 3 * w.shape[0]).astype(jnp.bfloat16)

    def tb(b):
        return b.reshape(1, -1)

    wDb2 = wDb[:, :, 0, 0].T.astype(jnp.bfloat16)
    wF2 = wF[:, :, 0, 0].T.astype(jnp.bfloat16)

    l = boxes[..., 0].reshape(A, K, 1)
    r = boxes[..., 1].reshape(A, K, 1)
    u = boxes[..., 2].reshape(A, K, 1)
    d = boxes[..., 3].reshape(A, K, 1)
    inva = 1.0 / ((d - u) * (r - l)).astype(jnp.float32)

    ji = jnp.arange(NOUT, dtype=jnp.int32)
    col = ji % BW
    central = (col >= 8) & (col < 96)
    hh = jnp.where(central, ji // BW, -1).reshape(1, NOUT).astype(jnp.int32)
    ww = (col - 8).reshape(1, NOUT).astype(jnp.int32)

    vspec = pl.BlockSpec(memory_space=pltpu.VMEM)
    desc = pl.pallas_call(
        _kernel,
        out_shape=jax.ShapeDtypeStruct((A, K, 256), jnp.float32),
        grid=(A, NS),
        in_specs=[
            pl.BlockSpec(memory_space=pl.ANY),
            pl.BlockSpec((1, K, 1), lambda a, s: (a, 0, 0)),
            pl.BlockSpec((1, K, 1), lambda a, s: (a, 0, 0)),
            pl.BlockSpec((1, K, 1), lambda a, s: (a, 0, 0)),
            pl.BlockSpec((1, K, 1), lambda a, s: (a, 0, 0)),
            pl.BlockSpec((1, K, 1), lambda a, s: (a, 0, 0)),
            pl.BlockSpec((1, NOUT), lambda a, s: (0, 0)),
            pl.BlockSpec((1, NOUT), lambda a, s: (0, 0)),
        ] + [pl.BlockSpec(memory_space=pltpu.VMEM)] * 18,
        out_specs=pl.BlockSpec((1, K, 256), lambda a, s: (a, 0, 0)),
        scratch_shapes=[
            pltpu.VMEM((BR, BW, CIN), jnp.bfloat16),
            pltpu.VMEM((FLAT, 256), jnp.bfloat16),
            pltpu.VMEM((FLAT, 256), jnp.bfloat16),
            pltpu.VMEM((FLAT, 768), jnp.bfloat16),
            pltpu.SemaphoreType.DMA,
        ],
        compiler_params=pltpu.CompilerParams(
            dimension_semantics=("parallel", "arbitrary"),
            vmem_limit_bytes=56 * 1024 * 1024,
        ),
        name="superbox_fused",
    )(feat, u, d, l, r, inva, hh, ww,
      tws(w1a), tb(b1a), tws(w1b), tb(b1b), tws(w2a), tb(b2a), tws(w2b), tb(b2b),
      tw3(w3a), tb(b3a), tw3(w3b), tb(b3b), tw3(wDa), tb(bDa),
      wDb2, tb(bDb), wF2, tb(bF))
    return desc
